# baseline (speedup 1.0000x reference)
import functools

import jax
import jax.numpy as jnp
from jax import lax
from jax.experimental import pallas as pl
from jax.experimental.pallas import tpu as pltpu
from jax.experimental.pallas import tpu_sc as plsc

N_NODES = 10000
N_EDGES = 320000
D = 128

NC = 2
NS = 16
NW = NC * NS
CHUNK = 125
CPW = 80
EPW = CPW * CHUNK
TOT_E = NW * EPW
ROWS_PAD = 10240
RPT = ROWS_PAD // NS

_mesh = plsc.VectorSubcoreMesh(core_axis_name="c", subcore_axis_name="s")


@functools.partial(
    pl.kernel,
    mesh=_mesh,
    out_type=jax.ShapeDtypeStruct((NC * ROWS_PAD, D), jnp.float32),
    scratch_types=[
        pltpu.VMEM((CPW // 2, CHUNK), jnp.int32),
        pltpu.VMEM((CPW // 2, CHUNK), jnp.int32),
        pltpu.VMEM((CHUNK, D), jnp.float32),
        pltpu.VMEM((CHUNK, D), jnp.float32),
        pltpu.VMEM_SHARED((ROWS_PAD, D), jnp.float32),
        pltpu.SemaphoreType.DMA,
        pltpu.SemaphoreType.DMA,
    ],
)
def _agg_kernel(x_hbm, src_hbm, dst_hbm, z_hbm, out_hbm,
                src_v, dst_v, rows_a, rows_b, agg_sh, sem_a, sem_b):
    c = lax.axis_index("c")
    s = lax.axis_index("s")
    wid = s * NC + c

    pltpu.sync_copy(z_hbm, agg_sh.at[pl.ds(s * RPT, RPT)])

    plsc.subcore_barrier()

    def gather(j, buf, sem):
        pltpu.async_copy(x_hbm.at[src_v.at[j]], buf, sem)

    def wait(j, buf, sem):
        pltpu.make_async_copy(x_hbm.at[src_v.at[j]], buf, sem).wait()

    def scatter(j, buf):
        pltpu.sync_copy(buf, agg_sh.at[dst_v.at[j]], add=True)

    HALF = CPW // 2
    for h in range(2):
        pltpu.sync_copy(src_hbm.at[pl.ds(wid * CPW + h * HALF, HALF)], src_v)
        pltpu.sync_copy(dst_hbm.at[pl.ds(wid * CPW + h * HALF, HALF)], dst_v)

        gather(0, rows_a, sem_a)

        def body(i, carry):
            j = 2 * i
            gather(j + 1, rows_b, sem_b)
            wait(j, rows_a, sem_a)
            scatter(j, rows_a)
            gather(j + 2, rows_a, sem_a)
            wait(j + 1, rows_b, sem_b)
            scatter(j + 1, rows_b)
            return carry

        lax.fori_loop(0, HALF // 2 - 1, body, 0)

        gather(HALF - 1, rows_b, sem_b)
        wait(HALF - 2, rows_a, sem_a)
        scatter(HALF - 2, rows_a)
        wait(HALF - 1, rows_b, sem_b)
        scatter(HALF - 1, rows_b)

    plsc.subcore_barrier()

    pltpu.sync_copy(agg_sh.at[pl.ds(s * RPT, RPT)],
                    out_hbm.at[pl.ds(c * ROWS_PAD + s * RPT, RPT)])


_BLK = 2048


def _mlp_body(eps_ref, x_ref, p0_ref, p1_ref, w1t_ref, b1_ref, w2t_ref,
              b2_ref, o_ref):
    scale = 1.0 + eps_ref[0]
    a = p0_ref[...] + p1_ref[...] + scale * x_ref[...]
    t = jnp.dot(a, w1t_ref[...], preferred_element_type=jnp.float32)
    t = t + b1_ref[...]
    o = jnp.dot(t, w2t_ref[...], preferred_element_type=jnp.float32)
    o_ref[...] = o + b2_ref[...]


_mlp = pl.pallas_call(
    _mlp_body,
    grid=(ROWS_PAD // _BLK,),
    in_specs=[
        pl.BlockSpec(memory_space=pltpu.SMEM),
        pl.BlockSpec((_BLK, D), lambda i: (i, 0)),
        pl.BlockSpec((_BLK, D), lambda i: (i, 0)),
        pl.BlockSpec((_BLK, D), lambda i: (i + ROWS_PAD // _BLK, 0)),
        pl.BlockSpec((D, D), lambda i: (0, 0)),
        pl.BlockSpec((1, D), lambda i: (0, 0)),
        pl.BlockSpec((D, D), lambda i: (0, 0)),
        pl.BlockSpec((1, D), lambda i: (0, 0)),
    ],
    out_specs=pl.BlockSpec((_BLK, D), lambda i: (i, 0)),
    out_shape=jax.ShapeDtypeStruct((ROWS_PAD, D), jnp.float32),
)


def kernel(x, adj_sparse, eps, W1, b1, W2, b2):
    src2d = adj_sparse[0].astype(jnp.int32).reshape(NW * CPW, CHUNK)
    dst2d = adj_sparse[1].astype(jnp.int32).reshape(NW * CPW, CHUNK)
    zeros = jnp.zeros((RPT, D), jnp.float32)

    parts = _agg_kernel(x, src2d, dst2d, zeros)

    h = _mlp(eps, x, parts, parts,
             W1.T, b1.reshape(1, D), W2.T, b2.reshape(1, D))
    return h[:N_NODES]


# speedup vs baseline: 1.0223x; 1.0223x over previous
import functools

import jax
import jax.numpy as jnp
from jax import lax
from jax.experimental import pallas as pl
from jax.experimental.pallas import tpu as pltpu
from jax.experimental.pallas import tpu_sc as plsc

N_NODES = 10000
N_EDGES = 320000
D = 128

NC = 2
NS = 16
NW = NC * NS
CHUNK = 128
CPW = 80
EPW = CPW * CHUNK
TOT_E = NW * EPW
ROWS_PAD = 10240
RPT = ROWS_PAD // NS

_mesh = plsc.VectorSubcoreMesh(core_axis_name="c", subcore_axis_name="s")


@functools.partial(
    pl.kernel,
    mesh=_mesh,
    out_type=jax.ShapeDtypeStruct((NC * ROWS_PAD, D), jnp.float32),
    scratch_types=[
        pltpu.VMEM((CPW // 2, CHUNK), jnp.int32),
        pltpu.VMEM((CPW // 2, CHUNK), jnp.int32),
        pltpu.VMEM((CHUNK, D), jnp.float32),
        pltpu.VMEM((CHUNK, D), jnp.float32),
        pltpu.VMEM_SHARED((ROWS_PAD, D), jnp.float32),
        pltpu.SemaphoreType.DMA,
        pltpu.SemaphoreType.DMA,
    ],
)
def _agg_kernel(x_hbm, src_hbm, dst_hbm, z_hbm, out_hbm,
                src_v, dst_v, rows_a, rows_b, agg_sh, sem_a, sem_b):
    c = lax.axis_index("c")
    s = lax.axis_index("s")
    wid = s * NC + c

    pltpu.sync_copy(z_hbm, agg_sh.at[pl.ds(s * RPT, RPT)])

    plsc.subcore_barrier()

    def gather(j, buf, sem):
        pltpu.async_copy(x_hbm.at[src_v.at[j]], buf, sem)

    def wait(j, buf, sem):
        pltpu.make_async_copy(x_hbm.at[src_v.at[j]], buf, sem).wait()

    def scatter(j, buf):
        pltpu.sync_copy(buf, agg_sh.at[dst_v.at[j]], add=True)

    HALF = CPW // 2
    for h in range(2):
        pltpu.sync_copy(src_hbm.at[pl.ds(wid * CPW + h * HALF, HALF)], src_v)
        pltpu.sync_copy(dst_hbm.at[pl.ds(wid * CPW + h * HALF, HALF)], dst_v)

        gather(0, rows_a, sem_a)

        def body(i, carry):
            j = 2 * i
            gather(j + 1, rows_b, sem_b)
            wait(j, rows_a, sem_a)
            scatter(j, rows_a)
            gather(j + 2, rows_a, sem_a)
            wait(j + 1, rows_b, sem_b)
            scatter(j + 1, rows_b)
            return carry

        lax.fori_loop(0, HALF // 2 - 1, body, 0)

        gather(HALF - 1, rows_b, sem_b)
        wait(HALF - 2, rows_a, sem_a)
        scatter(HALF - 2, rows_a)
        wait(HALF - 1, rows_b, sem_b)
        scatter(HALF - 1, rows_b)

    plsc.subcore_barrier()

    pltpu.sync_copy(agg_sh.at[pl.ds(s * RPT, RPT)],
                    out_hbm.at[pl.ds(c * ROWS_PAD + s * RPT, RPT)])


_BLK = 2048


def _mlp_body(eps_ref, x_ref, p0_ref, p1_ref, w1t_ref, b1_ref, w2t_ref,
              b2_ref, o_ref):
    scale = 1.0 + eps_ref[0]
    a = p0_ref[...] + p1_ref[...] + scale * x_ref[...]
    t = jnp.dot(a, w1t_ref[...], preferred_element_type=jnp.float32)
    t = t + b1_ref[...]
    o = jnp.dot(t, w2t_ref[...], preferred_element_type=jnp.float32)
    o_ref[...] = o + b2_ref[...]


_mlp = pl.pallas_call(
    _mlp_body,
    grid=(ROWS_PAD // _BLK,),
    in_specs=[
        pl.BlockSpec(memory_space=pltpu.SMEM),
        pl.BlockSpec((_BLK, D), lambda i: (i, 0)),
        pl.BlockSpec((_BLK, D), lambda i: (i, 0)),
        pl.BlockSpec((_BLK, D), lambda i: (i + ROWS_PAD // _BLK, 0)),
        pl.BlockSpec((D, D), lambda i: (0, 0)),
        pl.BlockSpec((1, D), lambda i: (0, 0)),
        pl.BlockSpec((D, D), lambda i: (0, 0)),
        pl.BlockSpec((1, D), lambda i: (0, 0)),
    ],
    out_specs=pl.BlockSpec((_BLK, D), lambda i: (i, 0)),
    out_shape=jax.ShapeDtypeStruct((ROWS_PAD, D), jnp.float32),
)


def kernel(x, adj_sparse, eps, W1, b1, W2, b2):
    src = adj_sparse[0].astype(jnp.int32)
    dst = adj_sparse[1].astype(jnp.int32)
    pad = TOT_E - N_EDGES
    pad_iota = lax.iota(jnp.int32, pad)
    src_p = jnp.concatenate([src, pad_iota % N_NODES])
    dst_p = jnp.concatenate([dst, N_NODES + pad_iota % (ROWS_PAD - N_NODES)])
    src2d = src_p.reshape(NW * CPW, CHUNK)
    dst2d = dst_p.reshape(NW * CPW, CHUNK)
    zeros = jnp.zeros((RPT, D), jnp.float32)

    parts = _agg_kernel(x, src2d, dst2d, zeros)

    h = _mlp(eps, x, parts, parts,
             W1.T, b1.reshape(1, D), W2.T, b2.reshape(1, D))
    return h[:N_NODES]
